# baseline (speedup 1.0000x reference)
import functools
import math

import jax
import jax.numpy as jnp
from jax import lax
from jax.experimental import pallas as pl
from jax.experimental.pallas import tpu as pltpu

EPS = 1e-06
THETA = 10000.0
CAUSAL_LOSS_WEIGHT = 0.01
HDIM = 64


def _score_kernel(hid_ref, post_ref, prior_ref, wr_ref, wc_ref,
                  br_ref, bc_ref, s_ref, c_ref, *, TI):
    h = hid_ref[0]
    d = post_ref[0] - prior_ref[0]
    surprise = jnp.sqrt(jnp.sum(d * d, axis=1, keepdims=True))
    rl = jnp.dot(h, wr_ref[...], preferred_element_type=jnp.float32)
    s = rl + surprise + br_ref[0, 0]
    s_ref[0, 0, :] = jnp.reshape(s, (TI,))
    cl = jnp.dot(h, wc_ref[...], preferred_element_type=jnp.float32)
    c_ref[0, 0, :] = jnp.reshape(cl + bc_ref[0, 0], (TI,))


def _select_kernel(s_ref, c_ref, pos_ref, selidx_ref, selpos_ref,
                   selsig_ref, bce_ref, *, T, K, TI):
    NTI = T // TI
    rvec = lax.broadcasted_iota(jnp.int32, (K, 1), 0)
    sel_i = jnp.zeros((K, 1), jnp.int32)
    sel_p = jnp.zeros((K, 1), jnp.int32)
    sel_v = jnp.zeros((K, 1), jnp.float32)
    bce_acc = jnp.zeros((1, TI), jnp.float32)
    ii = lax.broadcasted_iota(jnp.int32, (TI, TI), 0)
    jj = lax.broadcasted_iota(jnp.int32, (TI, TI), 1)

    for it in range(NTI):
        si = s_ref[0, :, pl.ds(it * TI, TI)]
        si_c = jnp.reshape(si, (TI, 1))

        def jbody(jt, acc, si_c=si_c, it=it):
            sj = s_ref[0, :, pl.ds(jt * TI, TI)]
            sjB = jnp.broadcast_to(sj, (TI, TI))
            siB = jnp.broadcast_to(si_c, (TI, TI))
            jg = jt * TI + jj
            ig = it * TI + ii
            beats = (sjB > siB) | ((sjB == siB) & (jg < ig))
            return acc + jnp.sum(beats.astype(jnp.int32), axis=1,
                                 keepdims=True)

        rank_c = lax.fori_loop(0, NTI, jbody, jnp.zeros((TI, 1), jnp.int32))
        rank_r = jnp.reshape(rank_c, (1, TI))

        eq = jnp.broadcast_to(rank_r, (K, TI)) == rvec
        jg_r = it * TI + lax.broadcasted_iota(jnp.int32, (K, TI), 1)
        sel_i = sel_i + jnp.sum(jnp.where(eq, jg_r, 0), axis=1, keepdims=True)
        pos_r = pos_ref[0, :, pl.ds(it * TI, TI)]
        sel_p = sel_p + jnp.sum(
            jnp.where(eq, jnp.broadcast_to(pos_r, (K, TI)), 0),
            axis=1, keepdims=True)
        sel_v = sel_v + jnp.sum(
            jnp.where(eq, jnp.broadcast_to(si, (K, TI)), 0.0),
            axis=1, keepdims=True)

        x = c_ref[0, :, pl.ds(it * TI, TI)]
        z = (rank_r < K).astype(jnp.float32)
        bce_acc = bce_acc + (jnp.maximum(x, 0.0) - x * z
                             + jnp.log(1.0 + jnp.exp(-jnp.abs(x))))

    selidx_ref[0, 0, :] = jnp.reshape(sel_i, (K,))
    selpos_ref[0, 0, :] = jnp.reshape(sel_p, (K,))
    sv = jnp.reshape(sel_v, (K,))
    selsig_ref[0, 0, :] = 1.0 / (1.0 + jnp.exp(-sv))
    bce_ref[0, 0, :] = jnp.full((128,), jnp.sum(bce_acc), jnp.float32)


def _gather_kernel(idx_ref, hid_ref, out_ref, *, K):
    b = pl.program_id(0)

    def body(r, carry):
        t = idx_ref[b, r]
        out_ref[0, pl.ds(r, 1), :] = hid_ref[0, pl.ds(t, 1), :]
        return carry

    lax.fori_loop(0, K, body, 0)


def _qkv_kernel(x_ref, pos_ref, ln1_ref, qw_ref, qb_ref, kw_ref, kb_ref,
                vw_ref, vb_ref, q_out, k_out, v_out, *, TN, H):
    x = x_ref[...]
    var = jnp.mean(x * x, axis=1, keepdims=True)
    hn = x * lax.rsqrt(var + EPS) * ln1_ref[...]

    q = jnp.dot(hn, qw_ref[...], preferred_element_type=jnp.float32) + qb_ref[...]
    k = jnp.dot(hn, kw_ref[...], preferred_element_type=jnp.float32) + kb_ref[...]
    v = jnp.dot(hn, vw_ref[...], preferred_element_type=jnp.float32) + vb_ref[...]

    pos = jnp.reshape(pos_ref[0], (TN, 1))
    half = HDIM // 2
    expo = lax.broadcasted_iota(jnp.int32, (1, half), 1).astype(jnp.float32) * (
        -2.0 * math.log(THETA) / HDIM)
    inv_freq = jnp.exp(expo)
    freqs = pos * inv_freq
    cos = jnp.cos(freqs)
    sin = jnp.sin(freqs)
    cos2 = jnp.concatenate([cos, cos], axis=1)[:, None, :]
    sin2 = jnp.concatenate([sin, sin], axis=1)[:, None, :]

    def rope(t):
        t3 = jnp.reshape(t, (TN, H, HDIM))
        t1 = t3[:, :, :half]
        t2 = t3[:, :, half:]
        rot = jnp.concatenate([-t2, t1], axis=2)
        out = t3 * cos2 + rot * sin2
        return jnp.reshape(out, (TN, H * HDIM))

    q_out[...] = rope(q)
    k_out[...] = rope(k)
    v_out[...] = v


def _attn_kernel(q_ref, k_ref, v_ref, o_ref, *, TN, N, H):
    i = pl.program_id(0)
    row = i * TN + lax.broadcasted_iota(jnp.int32, (TN, 1), 0)
    col = lax.broadcasted_iota(jnp.int32, (TN, N), 1)
    neg = jnp.finfo(jnp.float32).min
    bias = jnp.where(col <= row, 0.0, neg)
    scale = 1.0 / math.sqrt(HDIM)
    for h in range(H):
        sl = slice(h * HDIM, (h + 1) * HDIM)
        qh = q_ref[:, sl]
        kh = k_ref[:, sl]
        vh = v_ref[:, sl]
        s = lax.dot_general(qh, kh, (((1,), (1,)), ((), ())),
                            preferred_element_type=jnp.float32)
        s = s * scale + bias
        m = jnp.max(s, axis=1, keepdims=True)
        p = jnp.exp(s - m)
        den = jnp.sum(p, axis=1, keepdims=True)
        o = jnp.dot(p, vh, preferred_element_type=jnp.float32)
        o_ref[:, sl] = o / den


def _mlp_kernel(a_ref, x_ref, sig_ref, ow_ref, ln2_ref, gw_ref, uw_ref,
                dw_ref, out_ref, *, TN):
    x = x_ref[...]
    o = jnp.dot(a_ref[...], ow_ref[...], preferred_element_type=jnp.float32)
    h1 = x + o
    var = jnp.mean(h1 * h1, axis=1, keepdims=True)
    hn = h1 * lax.rsqrt(var + EPS) * ln2_ref[...]
    g = jnp.dot(hn, gw_ref[...], preferred_element_type=jnp.float32)
    u = jnp.dot(hn, uw_ref[...], preferred_element_type=jnp.float32)
    act = g / (1.0 + jnp.exp(-g)) * u
    mo = jnp.dot(act, dw_ref[...], preferred_element_type=jnp.float32)
    h2 = h1 + mo
    sig = jnp.reshape(sig_ref[0], (TN, 1))
    out_ref[...] = x + sig * (h2 - x)


def _scatter_kernel(idx_ref, hid_ref, upd_ref, out_ref, *, K):
    b = pl.program_id(0)
    out_ref[...] = hid_ref[...]

    def body(r, carry):
        t = idx_ref[b, r]
        out_ref[0, pl.ds(t, 1), :] = upd_ref[0, pl.ds(r, 1), :]
        return carry

    lax.fori_loop(0, K, body, 0)


def kernel(hidden_states, original, posterior, prior, w_r, b_r, w_c, b_c,
           ln1_w, q_w, q_b, k_w, k_b, v_w, v_b, o_w, ln2_w, gate_w, up_w,
           down_w, position_ids):
    B, T, D = hidden_states.shape
    K = max(1, T // 4)
    H = D // HDIM
    N = B * K
    F = gate_w.shape[1]
    TN = 256 if N % 256 == 0 else N
    NT = N // TN

    f32 = jnp.float32
    pos3 = position_ids.reshape(B, 1, T)

    TI = 256 if T % 256 == 0 else T
    NTI = T // TI
    scores, clogits = pl.pallas_call(
        functools.partial(_score_kernel, TI=TI),
        grid=(B, NTI),
        in_specs=[
            pl.BlockSpec((1, TI, D), lambda b, i: (b, i, 0)),
            pl.BlockSpec((1, TI, D), lambda b, i: (b, i, 0)),
            pl.BlockSpec((1, TI, D), lambda b, i: (b, i, 0)),
            pl.BlockSpec((D, 1), lambda b, i: (0, 0)),
            pl.BlockSpec((D, 1), lambda b, i: (0, 0)),
            pl.BlockSpec(memory_space=pltpu.SMEM),
            pl.BlockSpec(memory_space=pltpu.SMEM),
        ],
        out_specs=[
            pl.BlockSpec((1, 1, TI), lambda b, i: (b, 0, i)),
            pl.BlockSpec((1, 1, TI), lambda b, i: (b, 0, i)),
        ],
        out_shape=[
            jax.ShapeDtypeStruct((B, 1, T), f32),
            jax.ShapeDtypeStruct((B, 1, T), f32),
        ],
    )(hidden_states, posterior, prior, w_r, w_c,
      b_r.reshape(1, 1), b_c.reshape(1, 1))

    selidx, selpos, selsig, bce = pl.pallas_call(
        functools.partial(_select_kernel, T=T, K=K, TI=TI),
        grid=(B,),
        in_specs=[
            pl.BlockSpec((1, 1, T), lambda b: (b, 0, 0)),
            pl.BlockSpec((1, 1, T), lambda b: (b, 0, 0)),
            pl.BlockSpec((1, 1, T), lambda b: (b, 0, 0)),
        ],
        out_specs=[
            pl.BlockSpec((1, 1, K), lambda b: (b, 0, 0)),
            pl.BlockSpec((1, 1, K), lambda b: (b, 0, 0)),
            pl.BlockSpec((1, 1, K), lambda b: (b, 0, 0)),
            pl.BlockSpec((1, 1, 128), lambda b: (b, 0, 0)),
        ],
        out_shape=[
            jax.ShapeDtypeStruct((B, 1, K), jnp.int32),
            jax.ShapeDtypeStruct((B, 1, K), jnp.int32),
            jax.ShapeDtypeStruct((B, 1, K), f32),
            jax.ShapeDtypeStruct((B, 1, 128), f32),
        ],
    )(scores, clogits, pos3)

    aux_loss = (CAUSAL_LOSS_WEIGHT / (B * T)) * jnp.sum(bce[:, 0, 0])
    selidx2 = selidx.reshape(B, K)

    sel = pl.pallas_call(
        functools.partial(_gather_kernel, K=K),
        grid=(B,),
        in_specs=[
            pl.BlockSpec(memory_space=pltpu.SMEM),
            pl.BlockSpec((1, T, D), lambda b: (b, 0, 0)),
        ],
        out_specs=pl.BlockSpec((1, K, D), lambda b: (b, 0, 0)),
        out_shape=jax.ShapeDtypeStruct((B, K, D), f32),
    )(selidx2, hidden_states)
    sel2 = sel.reshape(N, D)

    posf = selpos.reshape(N).astype(f32).reshape(NT, 1, TN)
    sigf = selsig.reshape(N).reshape(NT, 1, TN)

    q, k, v = pl.pallas_call(
        functools.partial(_qkv_kernel, TN=TN, H=H),
        grid=(NT,),
        in_specs=[
            pl.BlockSpec((TN, D), lambda i: (i, 0)),
            pl.BlockSpec((1, 1, TN), lambda i: (i, 0, 0)),
            pl.BlockSpec((1, D), lambda i: (0, 0)),
            pl.BlockSpec((D, D), lambda i: (0, 0)),
            pl.BlockSpec((1, D), lambda i: (0, 0)),
            pl.BlockSpec((D, D), lambda i: (0, 0)),
            pl.BlockSpec((1, D), lambda i: (0, 0)),
            pl.BlockSpec((D, D), lambda i: (0, 0)),
            pl.BlockSpec((1, D), lambda i: (0, 0)),
        ],
        out_specs=[pl.BlockSpec((TN, D), lambda i: (i, 0))] * 3,
        out_shape=[jax.ShapeDtypeStruct((N, D), f32)] * 3,
    )(sel2, posf, ln1_w.reshape(1, D), q_w, q_b.reshape(1, D), k_w,
      k_b.reshape(1, D), v_w, v_b.reshape(1, D))

    attn = pl.pallas_call(
        functools.partial(_attn_kernel, TN=TN, N=N, H=H),
        grid=(NT,),
        in_specs=[
            pl.BlockSpec((TN, D), lambda i: (i, 0)),
            pl.BlockSpec((N, D), lambda i: (0, 0)),
            pl.BlockSpec((N, D), lambda i: (0, 0)),
        ],
        out_specs=pl.BlockSpec((TN, D), lambda i: (i, 0)),
        out_shape=jax.ShapeDtypeStruct((N, D), f32),
        compiler_params=pltpu.CompilerParams(
            vmem_limit_bytes=120 * 1024 * 1024),
    )(q, k, v)

    upd = pl.pallas_call(
        functools.partial(_mlp_kernel, TN=TN),
        grid=(NT,),
        in_specs=[
            pl.BlockSpec((TN, D), lambda i: (i, 0)),
            pl.BlockSpec((TN, D), lambda i: (i, 0)),
            pl.BlockSpec((1, 1, TN), lambda i: (i, 0, 0)),
            pl.BlockSpec((D, D), lambda i: (0, 0)),
            pl.BlockSpec((1, D), lambda i: (0, 0)),
            pl.BlockSpec((D, F), lambda i: (0, 0)),
            pl.BlockSpec((D, F), lambda i: (0, 0)),
            pl.BlockSpec((F, D), lambda i: (0, 0)),
        ],
        out_specs=pl.BlockSpec((TN, D), lambda i: (i, 0)),
        out_shape=jax.ShapeDtypeStruct((N, D), f32),
        compiler_params=pltpu.CompilerParams(
            vmem_limit_bytes=120 * 1024 * 1024),
    )(attn, sel2, sigf, o_w, ln2_w.reshape(1, D), gate_w, up_w, down_w)

    final = pl.pallas_call(
        functools.partial(_scatter_kernel, K=K),
        grid=(B,),
        in_specs=[
            pl.BlockSpec(memory_space=pltpu.SMEM),
            pl.BlockSpec((1, T, D), lambda b: (b, 0, 0)),
            pl.BlockSpec((1, K, D), lambda b: (b, 0, 0)),
        ],
        out_specs=pl.BlockSpec((1, T, D), lambda b: (b, 0, 0)),
        out_shape=jax.ShapeDtypeStruct((B, T, D), f32),
    )(selidx2, hidden_states, upd.reshape(B, K, D))

    return (final, aux_loss)
